# baseline (speedup 1.0000x reference)
import functools

import jax
import jax.numpy as jnp
from jax import lax
from jax.experimental import pallas as pl
from jax.experimental.pallas import tpu as pltpu
from jax.experimental.pallas import tpu_sc as plsc

N_NODES = 512
KSIZE = 16
N_CHAN = 8
N_STEPS = 2
NUM_SUBCORES = 16
ROWS_P1 = N_NODES // NUM_SUBCORES
ROWS_P2 = N_NODES // 32
N_GROUPS = N_NODES // 4


def _build_tables(x_v, tab_v):
    def body(th, carry):
        for u in range(2):
            g = 2 * th + u
            xs = [x_v[4 * g + i, :] for i in range(4)]
            tab_v[pl.ds(g * 256, 16)] = jnp.zeros((KSIZE,), jnp.float32)
            vals = {}
            for s in range(1, 16):
                kk = (s & -s).bit_length() - 1
                prev = s ^ (1 << kk)
                vals[s] = xs[kk] if prev == 0 else vals[prev] + xs[kk]
                tab_v[pl.ds(g * 256 + s * 16, 16)] = vals[s]
        return carry

    lax.fori_loop(0, N_GROUPS // 2, body, 0)


def _masked_rowsums(m_v, tab_v, row_off, nrows, blk=4):
    iota = lax.iota(jnp.int32, 16)
    rows = []

    for r0 in range(0, nrows, blk):

        def body(t, accs, r0=r0):
            out = []
            for ri in range(blk):
                r = row_off + r0 + ri
                w = plsc.bitcast(m_v[r, pl.ds(t * 64, 64)], jnp.int32)
                nibs = ((w & 1) | ((w >> 7) & 2)
                        | ((w >> 14) & 4) | ((w >> 21) & 8))
                acc = accs[ri]
                for k in range(16):
                    nb = nibs.at[jnp.full((16,), k, jnp.int32)].get(
                        mode="promise_in_bounds")
                    vidx = (nb << 4) + ((16 * t + k) << 8) + iota
                    acc = acc + plsc.load_gather(tab_v, [vidx])
                out.append(acc)
            return tuple(out)

        zero = jnp.zeros((KSIZE,), jnp.float32)
        accs = lax.fori_loop(0, N_NODES // 64, body,
                             tuple(zero for _ in range(blk)))
        rows.extend(accs)
    return rows


@functools.cache
def _build_call():
    mesh = plsc.VectorSubcoreMesh(core_axis_name="c", subcore_axis_name="s")

    @functools.partial(
        pl.kernel,
        out_type=jax.ShapeDtypeStruct((N_CHAN, N_NODES, KSIZE), jnp.float32),
        mesh=mesh,
        compiler_params=pltpu.CompilerParams(
            use_tc_tiling_on_sc=False, needs_layout_passes=False),
        scratch_types=[
            pltpu.VMEM((ROWS_P1, N_NODES), jnp.int8),
            pltpu.VMEM((N_NODES, KSIZE), jnp.float32),
            pltpu.VMEM((N_NODES, KSIZE), jnp.float32),
            pltpu.VMEM((N_GROUPS * 256,), jnp.float32),
            pltpu.VMEM((N_CHAN, N_STEPS, KSIZE), jnp.float32),
            pltpu.VMEM((ROWS_P1, KSIZE), jnp.float32),
            pltpu.VMEM((N_CHAN, ROWS_P2, KSIZE), jnp.float32),
            pltpu.VMEM_SHARED((N_NODES, KSIZE), jnp.float32),
            pltpu.SemaphoreType.DMA,
            pltpu.SemaphoreType.DMA,
            pltpu.SemaphoreType.DMA,
        ],
    )
    def wl_conv(m_hbm, l_hbm, k_hbm, out_hbm,
                m_v, x_v, p_v, tab_v, k_v, o1_v, o_v, sh_p,
                sem_a, sem_b, sem_c):
        sid = lax.axis_index("s")
        cid = lax.axis_index("c")
        base1 = sid * ROWS_P1
        cp_a = pltpu.async_copy(m_hbm.at[pl.ds(base1, ROWS_P1), :], m_v,
                                sem_a)
        cp_b = pltpu.async_copy(l_hbm, x_v, sem_b)
        cp_c = pltpu.async_copy(k_hbm, k_v, sem_c)
        cp_b.wait()
        _build_tables(x_v, tab_v)
        cp_a.wait()

        rows = _masked_rowsums(m_v, tab_v, 0, ROWS_P1)
        for r in range(ROWS_P1):
            o1_v[r, :] = rows[r]
        pltpu.sync_copy(o1_v, sh_p.at[pl.ds(base1, ROWS_P1), :])
        plsc.subcore_barrier()
        pltpu.sync_copy(sh_p, p_v)

        _build_tables(p_v, tab_v)
        row_off = cid * ROWS_P2
        base2 = base1 + row_off
        qs = _masked_rowsums(m_v, tab_v, row_off, ROWS_P2)
        cp_c.wait()
        for r in range(ROWS_P2):
            q = qs[r]
            p_i = p_v[base2 + r, :]
            l_i = x_v[base2 + r, :]
            for c in range(N_CHAN):
                k0 = k_v[c, 0, :]
                k1 = k_v[c, 1, :]
                o_v[c, r, :] = l_i + p_i * (k0 + k1) + q * (k0 * k1)
        for c in range(N_CHAN):
            pltpu.sync_copy(o_v.at[c],
                            out_hbm.at[c].at[pl.ds(base2, ROWS_P2), :])

    return wl_conv


def kernel(labelsList, ligand_structure, kernels):
    wl_conv = _build_call()
    return wl_conv(ligand_structure.astype(jnp.int8), labelsList, kernels)


# speedup vs baseline: 1.0494x; 1.0494x over previous
import functools

import jax
import jax.numpy as jnp
from jax import lax
from jax.experimental import pallas as pl
from jax.experimental.pallas import tpu as pltpu
from jax.experimental.pallas import tpu_sc as plsc

N_NODES = 512
KSIZE = 16
N_CHAN = 8
N_STEPS = 2
NUM_SUBCORES = 16
ROWS_P1 = N_NODES // NUM_SUBCORES
ROWS_P2 = N_NODES // 32
N_GROUPS = N_NODES // 4


def _build_tables(x_v, tab_v):
    def body(th, carry):
        for u in range(2):
            g = 2 * th + u
            xs = [x_v[4 * g + i, :] for i in range(4)]
            tab_v[pl.ds(g * 256, 16)] = jnp.zeros((KSIZE,), jnp.float32)
            vals = {}
            for s in range(1, 16):
                kk = (s & -s).bit_length() - 1
                prev = s ^ (1 << kk)
                vals[s] = xs[kk] if prev == 0 else vals[prev] + xs[kk]
                tab_v[pl.ds(g * 256 + s * 16, 16)] = vals[s]
        return carry

    lax.fori_loop(0, N_GROUPS // 2, body, 0)


def _masked_rowsums(m_v, tab_v, row_off, nrows, blk=4):
    iota = lax.iota(jnp.int32, 16)
    sh4 = iota & 3
    fold1 = iota ^ 1
    fold2 = iota ^ 2
    rows = []

    for r0 in range(0, nrows, blk):

        def body(t, accs, r0=r0):
            out = []
            for ri in range(blk):
                r = row_off + r0 + ri
                mrow = jnp.minimum(m_v[r, pl.ds(t * 16, 16)], 1)
                sh = mrow << sh4
                s1 = sh + sh.at[fold1].get(mode="promise_in_bounds")
                nib = s1 + s1.at[fold2].get(mode="promise_in_bounds")
                acc = accs[ri]
                for k in range(4):
                    nb = nib.at[jnp.full((16,), 4 * k, jnp.int32)].get(
                        mode="promise_in_bounds")
                    vidx = (nb << 4) + ((4 * t + k) << 8) + iota
                    acc = acc + plsc.load_gather(tab_v, [vidx])
                out.append(acc)
            return tuple(out)

        zero = jnp.zeros((KSIZE,), jnp.float32)
        accs = lax.fori_loop(0, N_NODES // 16, body,
                             tuple(zero for _ in range(blk)))
        rows.extend(accs)
    return rows


@functools.cache
def _build_call():
    mesh = plsc.VectorSubcoreMesh(core_axis_name="c", subcore_axis_name="s")

    @functools.partial(
        pl.kernel,
        out_type=jax.ShapeDtypeStruct((N_CHAN, N_NODES, KSIZE), jnp.float32),
        mesh=mesh,
        compiler_params=pltpu.CompilerParams(
            use_tc_tiling_on_sc=False, needs_layout_passes=False),
        scratch_types=[
            pltpu.VMEM((ROWS_P1, N_NODES), jnp.int32),
            pltpu.VMEM((N_NODES, KSIZE), jnp.float32),
            pltpu.VMEM((N_NODES, KSIZE), jnp.float32),
            pltpu.VMEM((N_GROUPS * 256,), jnp.float32),
            pltpu.VMEM((N_CHAN, N_STEPS, KSIZE), jnp.float32),
            pltpu.VMEM((ROWS_P1, KSIZE), jnp.float32),
            pltpu.VMEM((N_CHAN, ROWS_P2, KSIZE), jnp.float32),
            pltpu.VMEM_SHARED((N_NODES, KSIZE), jnp.float32),
            pltpu.SemaphoreType.DMA,
            pltpu.SemaphoreType.DMA,
            pltpu.SemaphoreType.DMA,
        ],
    )
    def wl_conv(m_hbm, l_hbm, k_hbm, out_hbm,
                m_v, x_v, p_v, tab_v, k_v, o1_v, o_v, sh_p,
                sem_a, sem_b, sem_c):
        sid = lax.axis_index("s")
        cid = lax.axis_index("c")
        base1 = sid * ROWS_P1
        cp_a = pltpu.async_copy(m_hbm.at[pl.ds(base1, ROWS_P1), :], m_v,
                                sem_a)
        cp_b = pltpu.async_copy(l_hbm, x_v, sem_b)
        cp_c = pltpu.async_copy(k_hbm, k_v, sem_c)
        cp_b.wait()
        _build_tables(x_v, tab_v)
        cp_a.wait()

        rows = _masked_rowsums(m_v, tab_v, 0, ROWS_P1)
        for r in range(ROWS_P1):
            o1_v[r, :] = rows[r]
        pltpu.sync_copy(o1_v, sh_p.at[pl.ds(base1, ROWS_P1), :])
        plsc.subcore_barrier()
        pltpu.sync_copy(sh_p, p_v)

        _build_tables(p_v, tab_v)
        row_off = cid * ROWS_P2
        base2 = base1 + row_off
        qs = _masked_rowsums(m_v, tab_v, row_off, ROWS_P2)
        cp_c.wait()
        for r in range(ROWS_P2):
            q = qs[r]
            p_i = p_v[base2 + r, :]
            l_i = x_v[base2 + r, :]
            for c in range(N_CHAN):
                k0 = k_v[c, 0, :]
                k1 = k_v[c, 1, :]
                o_v[c, r, :] = l_i + p_i * (k0 + k1) + q * (k0 * k1)
        for c in range(N_CHAN):
            pltpu.sync_copy(o_v.at[c],
                            out_hbm.at[c].at[pl.ds(base2, ROWS_P2), :])

    return wl_conv


def kernel(labelsList, ligand_structure, kernels):
    wl_conv = _build_call()
    return wl_conv(ligand_structure, labelsList, kernels)
